# baseline (speedup 1.0000x reference)
import jax
import jax.numpy as jnp
from jax.experimental import pallas as pl

_B, _P, _K = 32, 512, 20
_NK = _B * _P * _K
_INF = float(jnp.inf)


def _gather_cols(tbl, j):
    f = tbl.shape[0]
    jb = jnp.broadcast_to(j[None, :], (f, _P))
    jc = jb & 127
    out = None
    for c in range(_P // 128):
        g = jnp.take_along_axis(tbl[:, c * 128:(c + 1) * 128], jc, axis=1,
                                mode='promise_in_bounds')
        out = g if out is None else jnp.where((jb >> 7) == c, g, out)
    return out


def _pop_min(dm, iota2_rf):
    m = jnp.min(dm, axis=0, keepdims=True)
    eq = dm == m
    eqf = jnp.where(eq, 1.0, 0.0)
    hl = jax.lax.dot_general(iota2_rf, eqf, (((1,), (0,)), ((), ())),
                             preferred_element_type=jnp.float32)
    j = (hl[0:1] * 32.0 + hl[1:2]).astype(jnp.int32)[0]
    dm = jnp.where(eq, _INF, dm)
    return dm, j


def _iota2_rf():
    ii = jax.lax.broadcasted_iota(jnp.int32, (1, _P), 1)
    return jnp.concatenate([ii >> 5, ii & 31], axis=0).astype(jnp.float32)


def _pairwise(xT):
    sq = jnp.sum(xT * xT, axis=0, keepdims=True)
    g = jax.lax.dot_general(xT, xT, (((0,), (0,)), ((), ())),
                            preferred_element_type=jnp.float32)
    return sq + jnp.transpose(sq) - 2.0 * g


def _proj_base(xT, w1aT, w1bT, b1c):
    pdT = jnp.dot(w1bT, xT, preferred_element_type=jnp.float32)
    baseT = jnp.dot(w1aT, xT, preferred_element_type=jnp.float32) - pdT + b1c
    return pdT, baseT


def _c1_body(posT_ref, w1aT_ref, w1bT_ref, b1c_ref, h1_ref, s_ref, q_ref):
    xT = posT_ref[0]
    dm = _pairwise(xT)
    pdT, baseT = _proj_base(xT, w1aT_ref[...], w1bT_ref[...], b1c_ref[...])
    iota2 = _iota2_rf()
    ssum = jnp.zeros((64, 1), jnp.float32)
    qsum = jnp.zeros((64, 1), jnp.float32)
    for k in range(_K):
        dm, j = _pop_min(dm, iota2)
        h1 = baseT + _gather_cols(pdT, j)
        h1_ref[0, :, k * _P:(k + 1) * _P] = h1
        ssum += jnp.sum(h1, axis=1, keepdims=True)
        qsum += jnp.sum(h1 * h1, axis=1, keepdims=True)
    s_ref[0] = ssum.T
    q_ref[0] = qsum.T


def _bn_fold(s_ref, q_ref, g_ref, be_ref):
    s = jnp.sum(s_ref[...].reshape(_B, 64), axis=0, keepdims=True) / _NK
    q = jnp.sum(q_ref[...].reshape(_B, 64), axis=0, keepdims=True) / _NK
    var = q - s * s
    sc = g_ref[...] * jax.lax.rsqrt(var + 1e-5)
    sh = be_ref[...] - s * sc
    return jnp.transpose(sc), jnp.transpose(sh)


def _c2_body(h1_ref, s1_ref, q1_ref, g1_ref, be1_ref, w2T_ref, b2c_ref,
             h2_ref, s_ref, q_ref):
    sc1, sh1 = _bn_fold(s1_ref, q1_ref, g1_ref, be1_ref)
    a1 = jnp.maximum(h1_ref[0] * sc1 + sh1, 0.0)
    h2 = (jnp.dot(w2T_ref[...], a1, preferred_element_type=jnp.float32)
          + b2c_ref[...])
    h2_ref[0] = h2
    s_ref[0] = jnp.sum(h2, axis=1, keepdims=True).T
    q_ref[0] = jnp.sum(h2 * h2, axis=1, keepdims=True).T


def _c3_body(h2_ref, s2_ref, q2_ref, g2_ref, be2_ref, w3T_ref, b3c_ref,
             w2aT_ref, w2bT_ref, c2bc_ref, l1aT_ref, l1bT_ref, bl1c_ref,
             pool_ref):
    sc2, sh2 = _bn_fold(s2_ref, q2_ref, g2_ref, be2_ref)
    a2 = jnp.maximum(h2_ref[0] * sc2 + sh2, 0.0)
    h3 = (jnp.dot(w3T_ref[...], a2, preferred_element_type=jnp.float32)
          + b3c_ref[...])
    x1T = h3[:, 0:_P]
    for k in range(1, _K):
        x1T = jnp.maximum(x1T, h3[:, k * _P:(k + 1) * _P])
    dm2 = _pairwise(x1T)
    bdT = jnp.dot(w2bT_ref[...], x1T, preferred_element_type=jnp.float32)
    base2 = (jnp.dot(w2aT_ref[...], x1T, preferred_element_type=jnp.float32)
             - bdT + c2bc_ref[...])
    iota2 = _iota2_rf()
    mm = jnp.full((128, _P), -_INF, jnp.float32)
    for k in range(_K):
        dm2, j = _pop_min(dm2, iota2)
        mm = jnp.maximum(mm, _gather_cols(bdT, j))
    x2T = base2 + mm
    outT = (jnp.dot(l1aT_ref[...], x1T, preferred_element_type=jnp.float32)
            + jnp.dot(l1bT_ref[...], x2T, preferred_element_type=jnp.float32)
            + bl1c_ref[...])
    pool_ref[0] = jnp.max(outT, axis=1, keepdims=True).T


def _head_body(x_ref, w1_ref, b1_ref, w2_ref, b2_ref, w3_ref, b3_ref, o_ref):
    x = x_ref[...]
    h = jnp.maximum(jnp.dot(x, w1_ref[...], preferred_element_type=jnp.float32)
                    + b1_ref[...], 0.0)
    h = jnp.maximum(jnp.dot(h, w2_ref[...], preferred_element_type=jnp.float32)
                    + b2_ref[...], 0.0)
    z = (jnp.dot(h, w3_ref[...], preferred_element_type=jnp.float32)
         + b3_ref[...])
    zm = z - jnp.max(z, axis=1, keepdims=True)
    o_ref[...] = zm - jnp.log(jnp.sum(jnp.exp(zm), axis=1, keepdims=True))


def _full(shape):
    return pl.BlockSpec(shape, lambda b: (0,) * len(shape))


def _perb(shape):
    return pl.BlockSpec((1,) + shape, lambda b: (b,) + (0,) * len(shape))


def kernel(pos, batch, params):
    p = params
    posT = jnp.transpose(pos.reshape(_B, _P, 3), (0, 2, 1))
    w1aT = p['c1_w1'][:3].T
    w1bT = p['c1_w1'][3:].T
    b1c = p['c1_b1'].reshape(64, 1)
    g1 = p['c1_g1'].reshape(1, 64)
    be1 = p['c1_be1'].reshape(1, 64)
    w2T = p['c1_w2'].T
    b2c = p['c1_b2'].reshape(64, 1)
    g2 = p['c1_g2'].reshape(1, 64)
    be2 = p['c1_be2'].reshape(1, 64)
    w3T = p['c1_w3'].T
    b3c = p['c1_b3'].reshape(64, 1)
    w2aT = p['c2_w1'][:64].T
    w2bT = p['c2_w1'][64:].T
    c2bc = p['c2_b1'].reshape(128, 1)
    l1aT = p['lin1_w'][:64].T
    l1bT = p['lin1_w'][64:].T
    bl1c = p['lin1_b'].reshape(1024, 1)

    f32 = jnp.float32
    h1, s1, q1 = pl.pallas_call(
        _c1_body,
        grid=(_B,),
        in_specs=[_perb((3, _P)), _full((64, 3)), _full((64, 3)),
                  _full((64, 1))],
        out_specs=(_perb((64, _K * _P)), _perb((1, 64)), _perb((1, 64))),
        out_shape=(jax.ShapeDtypeStruct((_B, 64, _K * _P), f32),
                   jax.ShapeDtypeStruct((_B, 1, 64), f32),
                   jax.ShapeDtypeStruct((_B, 1, 64), f32)),
    )(posT, w1aT, w1bT, b1c)

    h2, s2, q2 = pl.pallas_call(
        _c2_body,
        grid=(_B,),
        in_specs=[_perb((64, _K * _P)), _full((_B, 1, 64)),
                  _full((_B, 1, 64)), _full((1, 64)), _full((1, 64)),
                  _full((64, 64)), _full((64, 1))],
        out_specs=(_perb((64, _K * _P)), _perb((1, 64)), _perb((1, 64))),
        out_shape=(jax.ShapeDtypeStruct((_B, 64, _K * _P), f32),
                   jax.ShapeDtypeStruct((_B, 1, 64), f32),
                   jax.ShapeDtypeStruct((_B, 1, 64), f32)),
    )(h1, s1, q1, g1, be1, w2T, b2c)

    pooled = pl.pallas_call(
        _c3_body,
        grid=(_B,),
        in_specs=[_perb((64, _K * _P)), _full((_B, 1, 64)),
                  _full((_B, 1, 64)), _full((1, 64)), _full((1, 64)),
                  _full((64, 64)), _full((64, 1)), _full((128, 64)),
                  _full((128, 64)), _full((128, 1)), _full((1024, 64)),
                  _full((1024, 128)), _full((1024, 1))],
        out_specs=_perb((1, 1024)),
        out_shape=jax.ShapeDtypeStruct((_B, 1, 1024), f32),
    )(h2, s2, q2, g2, be2, w3T, b3c, w2aT, w2bT, c2bc, l1aT, l1bT, bl1c)

    out = pl.pallas_call(
        _head_body,
        in_specs=[pl.BlockSpec((_B, 1024), lambda: (0, 0)),
                  pl.BlockSpec((1024, 512), lambda: (0, 0)),
                  pl.BlockSpec((1, 512), lambda: (0, 0)),
                  pl.BlockSpec((512, 256), lambda: (0, 0)),
                  pl.BlockSpec((1, 256), lambda: (0, 0)),
                  pl.BlockSpec((256, 40), lambda: (0, 0)),
                  pl.BlockSpec((1, 40), lambda: (0, 0))],
        out_specs=pl.BlockSpec((_B, 40), lambda: (0, 0)),
        out_shape=jax.ShapeDtypeStruct((_B, 40), f32),
    )(pooled.reshape(_B, 1024), p['m_w1'], p['m_b1'].reshape(1, 512),
      p['m_w2'], p['m_b2'].reshape(1, 256), p['m_w3'],
      p['m_b3'].reshape(1, 40))
    return out


# speedup vs baseline: 2.5527x; 2.5527x over previous
import jax
import jax.numpy as jnp
from jax.experimental import pallas as pl

_B, _P, _K = 32, 512, 20
_G = 1
_NK = _B * _P * _K
_INF = float(jnp.inf)


def _pop_eq(dm):
    m = jnp.min(dm, axis=0, keepdims=True)
    eq = dm == m
    eqf = jnp.where(eq, 1.0, 0.0)
    return jnp.where(eq, _INF, dm), eqf


def _onehot_gather(tbl, eqf):
    return jax.lax.dot_general(tbl, eqf, (((1,), (0,)), ((), ())),
                               preferred_element_type=jnp.float32)


def _pairwise(xT):
    sq = jnp.sum(xT * xT, axis=0, keepdims=True)
    g = jax.lax.dot_general(xT, xT, (((0,), (0,)), ((), ())),
                            preferred_element_type=jnp.float32)
    return sq + jnp.transpose(sq) - 2.0 * g


def _proj_base(xT, w1aT, w1bT, b1c):
    pdT = jnp.dot(w1bT, xT, preferred_element_type=jnp.float32)
    baseT = jnp.dot(w1aT, xT, preferred_element_type=jnp.float32) - pdT + b1c
    return pdT, baseT


def _c1_body(posT_ref, w1aT_ref, w1bT_ref, b1c_ref, h1_ref, s_ref, q_ref):
    pds, bases, dms = [], [], []
    for g in range(_G):
        xT = posT_ref[g]
        dms.append(_pairwise(xT))
        pdT, baseT = _proj_base(xT, w1aT_ref[...], w1bT_ref[...],
                                b1c_ref[...])
        pds.append(pdT)
        bases.append(baseT)
    ssum = [jnp.zeros((64, 1), jnp.float32) for _ in range(_G)]
    qsum = [jnp.zeros((64, 1), jnp.float32) for _ in range(_G)]
    for k in range(_K):
        for g in range(_G):
            dms[g], eqf = _pop_eq(dms[g])
            h1 = bases[g] + _onehot_gather(pds[g], eqf)
            h1_ref[g, :, k * _P:(k + 1) * _P] = h1
            ssum[g] += jnp.sum(h1, axis=1, keepdims=True)
            qsum[g] += jnp.sum(h1 * h1, axis=1, keepdims=True)
    for g in range(_G):
        s_ref[g] = ssum[g].T
        q_ref[g] = qsum[g].T


def _bn_fold(s_ref, q_ref, g_ref, be_ref):
    s = jnp.sum(s_ref[...].reshape(_B, 64), axis=0, keepdims=True) / _NK
    q = jnp.sum(q_ref[...].reshape(_B, 64), axis=0, keepdims=True) / _NK
    var = q - s * s
    sc = g_ref[...] * jax.lax.rsqrt(var + 1e-5)
    sh = be_ref[...] - s * sc
    return jnp.transpose(sc), jnp.transpose(sh)


def _c2_body(h1_ref, s1_ref, q1_ref, g1_ref, be1_ref, w2T_ref, b2c_ref,
             h2_ref, s_ref, q_ref):
    sc1, sh1 = _bn_fold(s1_ref, q1_ref, g1_ref, be1_ref)
    a1 = jnp.maximum(h1_ref[0] * sc1 + sh1, 0.0)
    h2 = (jnp.dot(w2T_ref[...], a1, preferred_element_type=jnp.float32)
          + b2c_ref[...])
    h2_ref[0] = h2
    s_ref[0] = jnp.sum(h2, axis=1, keepdims=True).T
    q_ref[0] = jnp.sum(h2 * h2, axis=1, keepdims=True).T


def _c3_body(h2_ref, s2_ref, q2_ref, g2_ref, be2_ref, w3T_ref, b3c_ref,
             w2aT_ref, w2bT_ref, c2bc_ref, l1aT_ref, l1bT_ref, bl1c_ref,
             pool_ref):
    sc2, sh2 = _bn_fold(s2_ref, q2_ref, g2_ref, be2_ref)
    x1s, dms, bds, b2s = [], [], [], []
    for g in range(_G):
        a2 = jnp.maximum(h2_ref[g] * sc2 + sh2, 0.0)
        h3 = (jnp.dot(w3T_ref[...], a2, preferred_element_type=jnp.float32)
              + b3c_ref[...])
        x1T = h3[:, 0:_P]
        for k in range(1, _K):
            x1T = jnp.maximum(x1T, h3[:, k * _P:(k + 1) * _P])
        x1s.append(x1T)
        dms.append(_pairwise(x1T))
        bdT = jnp.dot(w2bT_ref[...], x1T, preferred_element_type=jnp.float32)
        bds.append(bdT)
        b2s.append(jnp.dot(w2aT_ref[...], x1T,
                           preferred_element_type=jnp.float32)
                   - bdT + c2bc_ref[...])
    mms = [jnp.full((128, _P), -_INF, jnp.float32) for _ in range(_G)]
    for k in range(_K):
        for g in range(_G):
            dms[g], eqf = _pop_eq(dms[g])
            mms[g] = jnp.maximum(mms[g], _onehot_gather(bds[g], eqf))
    for g in range(_G):
        x2T = b2s[g] + mms[g]
        outT = (jnp.dot(l1aT_ref[...], x1s[g],
                        preferred_element_type=jnp.float32)
                + jnp.dot(l1bT_ref[...], x2T,
                          preferred_element_type=jnp.float32)
                + bl1c_ref[...])
        pool_ref[g] = jnp.max(outT, axis=1, keepdims=True).T


def _head_body(x_ref, w1_ref, b1_ref, w2_ref, b2_ref, w3_ref, b3_ref, o_ref):
    x = x_ref[...]
    h = jnp.maximum(jnp.dot(x, w1_ref[...], preferred_element_type=jnp.float32)
                    + b1_ref[...], 0.0)
    h = jnp.maximum(jnp.dot(h, w2_ref[...], preferred_element_type=jnp.float32)
                    + b2_ref[...], 0.0)
    z = (jnp.dot(h, w3_ref[...], preferred_element_type=jnp.float32)
         + b3_ref[...])
    zm = z - jnp.max(z, axis=1, keepdims=True)
    o_ref[...] = zm - jnp.log(jnp.sum(jnp.exp(zm), axis=1, keepdims=True))


def _full(shape):
    return pl.BlockSpec(shape, lambda b: (0,) * len(shape))


def _perb(shape):
    return pl.BlockSpec((1,) + shape, lambda b: (b,) + (0,) * len(shape))


def _perg(shape):
    return pl.BlockSpec((_G,) + shape, lambda b: (b,) + (0,) * len(shape))


def kernel(pos, batch, params):
    p = params
    posT = jnp.transpose(pos.reshape(_B, _P, 3), (0, 2, 1))
    w1aT = p['c1_w1'][:3].T
    w1bT = p['c1_w1'][3:].T
    b1c = p['c1_b1'].reshape(64, 1)
    g1 = p['c1_g1'].reshape(1, 64)
    be1 = p['c1_be1'].reshape(1, 64)
    w2T = p['c1_w2'].T
    b2c = p['c1_b2'].reshape(64, 1)
    g2 = p['c1_g2'].reshape(1, 64)
    be2 = p['c1_be2'].reshape(1, 64)
    w3T = p['c1_w3'].T
    b3c = p['c1_b3'].reshape(64, 1)
    w2aT = p['c2_w1'][:64].T
    w2bT = p['c2_w1'][64:].T
    c2bc = p['c2_b1'].reshape(128, 1)
    l1aT = p['lin1_w'][:64].T
    l1bT = p['lin1_w'][64:].T
    bl1c = p['lin1_b'].reshape(1024, 1)

    f32 = jnp.float32
    h1, s1, q1 = pl.pallas_call(
        _c1_body,
        grid=(_B // _G,),
        in_specs=[_perg((3, _P)), _full((64, 3)), _full((64, 3)),
                  _full((64, 1))],
        out_specs=(_perg((64, _K * _P)), _perg((1, 64)), _perg((1, 64))),
        out_shape=(jax.ShapeDtypeStruct((_B, 64, _K * _P), f32),
                   jax.ShapeDtypeStruct((_B, 1, 64), f32),
                   jax.ShapeDtypeStruct((_B, 1, 64), f32)),
    )(posT, w1aT, w1bT, b1c)

    h2, s2, q2 = pl.pallas_call(
        _c2_body,
        grid=(_B,),
        in_specs=[_perb((64, _K * _P)), _full((_B, 1, 64)),
                  _full((_B, 1, 64)), _full((1, 64)), _full((1, 64)),
                  _full((64, 64)), _full((64, 1))],
        out_specs=(_perb((64, _K * _P)), _perb((1, 64)), _perb((1, 64))),
        out_shape=(jax.ShapeDtypeStruct((_B, 64, _K * _P), f32),
                   jax.ShapeDtypeStruct((_B, 1, 64), f32),
                   jax.ShapeDtypeStruct((_B, 1, 64), f32)),
    )(h1, s1, q1, g1, be1, w2T, b2c)

    pooled = pl.pallas_call(
        _c3_body,
        grid=(_B // _G,),
        in_specs=[_perg((64, _K * _P)), _full((_B, 1, 64)),
                  _full((_B, 1, 64)), _full((1, 64)), _full((1, 64)),
                  _full((64, 64)), _full((64, 1)), _full((128, 64)),
                  _full((128, 64)), _full((128, 1)), _full((1024, 64)),
                  _full((1024, 128)), _full((1024, 1))],
        out_specs=_perg((1, 1024)),
        out_shape=jax.ShapeDtypeStruct((_B, 1, 1024), f32),
    )(h2, s2, q2, g2, be2, w3T, b3c, w2aT, w2bT, c2bc, l1aT, l1bT, bl1c)

    out = pl.pallas_call(
        _head_body,
        in_specs=[pl.BlockSpec((_B, 1024), lambda: (0, 0)),
                  pl.BlockSpec((1024, 512), lambda: (0, 0)),
                  pl.BlockSpec((1, 512), lambda: (0, 0)),
                  pl.BlockSpec((512, 256), lambda: (0, 0)),
                  pl.BlockSpec((1, 256), lambda: (0, 0)),
                  pl.BlockSpec((256, 40), lambda: (0, 0)),
                  pl.BlockSpec((1, 40), lambda: (0, 0))],
        out_specs=pl.BlockSpec((_B, 40), lambda: (0, 0)),
        out_shape=jax.ShapeDtypeStruct((_B, 40), f32),
    )(pooled.reshape(_B, 1024), p['m_w1'], p['m_b1'].reshape(1, 512),
      p['m_w2'], p['m_b2'].reshape(1, 256), p['m_w3'],
      p['m_b3'].reshape(1, 40))
    return out


# speedup vs baseline: 2.5559x; 1.0012x over previous
import jax
import jax.numpy as jnp
from jax.experimental import pallas as pl

_B, _P, _K = 32, 512, 20
_G1 = 1
_G3 = 1
_NK = _B * _P * _K
_INF = float(jnp.inf)


def _pop_eq(dm):
    m = jnp.min(dm, axis=0, keepdims=True)
    eq = dm == m
    eqf = jnp.where(eq, 1.0, 0.0)
    return jnp.where(eq, _INF, dm), eqf


def _onehot_gather(tbl, eqf):
    return jax.lax.dot_general(tbl, eqf, (((1,), (0,)), ((), ())),
                               preferred_element_type=jnp.float32)


def _pairwise(xT):
    sq = jnp.sum(xT * xT, axis=0, keepdims=True)
    g = jax.lax.dot_general(xT, xT, (((0,), (0,)), ((), ())),
                            preferred_element_type=jnp.float32)
    return sq + jnp.transpose(sq) - 2.0 * g


def _proj_base(xT, w1aT, w1bT, b1c):
    pdT = jnp.dot(w1bT, xT, preferred_element_type=jnp.float32)
    baseT = jnp.dot(w1aT, xT, preferred_element_type=jnp.float32) - pdT + b1c
    return pdT, baseT


def _c1_body(posT_ref, w1aT_ref, w1bT_ref, b1c_ref, h1_ref, s_ref, q_ref):
    pds, bases, dms = [], [], []
    for g in range(_G1):
        xT = posT_ref[g]
        dms.append(_pairwise(xT))
        pdT, baseT = _proj_base(xT, w1aT_ref[...], w1bT_ref[...],
                                b1c_ref[...])
        pds.append(pdT)
        bases.append(baseT)
    ssum = [jnp.zeros((64, 1), jnp.float32) for _ in range(_G1)]
    qsum = [jnp.zeros((64, 1), jnp.float32) for _ in range(_G1)]
    for k in range(_K):
        for g in range(_G1):
            dms[g], eqf = _pop_eq(dms[g])
            h1 = bases[g] + _onehot_gather(pds[g], eqf)
            h1_ref[g, :, k * _P:(k + 1) * _P] = h1
            ssum[g] += jnp.sum(h1, axis=1, keepdims=True)
            qsum[g] += jnp.sum(h1 * h1, axis=1, keepdims=True)
    for g in range(_G1):
        s_ref[g] = ssum[g].T
        q_ref[g] = qsum[g].T


def _bn_fold(s_ref, q_ref, g_ref, be_ref):
    s = jnp.sum(s_ref[...].reshape(_B, 64), axis=0, keepdims=True) / _NK
    q = jnp.sum(q_ref[...].reshape(_B, 64), axis=0, keepdims=True) / _NK
    var = q - s * s
    sc = g_ref[...] * jax.lax.rsqrt(var + 1e-5)
    sh = be_ref[...] - s * sc
    return jnp.transpose(sc), jnp.transpose(sh)


def _c2_body(h1_ref, s1_ref, q1_ref, g1_ref, be1_ref, w2T_ref, b2c_ref,
             h2_ref, s_ref, q_ref):
    sc1, sh1 = _bn_fold(s1_ref, q1_ref, g1_ref, be1_ref)
    a1 = jnp.maximum(h1_ref[0] * sc1 + sh1, 0.0)
    h2 = (jnp.dot(w2T_ref[...], a1, preferred_element_type=jnp.float32)
          + b2c_ref[...])
    h2_ref[0] = h2
    s_ref[0] = jnp.sum(h2, axis=1, keepdims=True).T
    q_ref[0] = jnp.sum(h2 * h2, axis=1, keepdims=True).T


def _c3_body(h2_ref, s2_ref, q2_ref, g2_ref, be2_ref, w3T_ref, b3c_ref,
             w2aT_ref, w2bT_ref, c2bc_ref, l1aT_ref, l1bT_ref, bl1c_ref,
             pool_ref):
    sc2, sh2 = _bn_fold(s2_ref, q2_ref, g2_ref, be2_ref)
    x1s, dms, bds, b2s = [], [], [], []
    for g in range(_G3):
        a2 = jnp.maximum(h2_ref[g] * sc2 + sh2, 0.0)
        h3 = (jnp.dot(w3T_ref[...], a2, preferred_element_type=jnp.float32)
              + b3c_ref[...])
        x1T = h3[:, 0:_P]
        for k in range(1, _K):
            x1T = jnp.maximum(x1T, h3[:, k * _P:(k + 1) * _P])
        x1s.append(x1T)
        dms.append(_pairwise(x1T))
        bdT = jnp.dot(w2bT_ref[...], x1T, preferred_element_type=jnp.float32)
        bds.append(bdT)
        b2s.append(jnp.dot(w2aT_ref[...], x1T,
                           preferred_element_type=jnp.float32)
                   - bdT + c2bc_ref[...])
    mms = [jnp.full((128, _P), -_INF, jnp.float32) for _ in range(_G3)]
    for k in range(_K):
        for g in range(_G3):
            dms[g], eqf = _pop_eq(dms[g])
            mms[g] = jnp.maximum(mms[g], _onehot_gather(bds[g], eqf))
    for g in range(_G3):
        x2T = b2s[g] + mms[g]
        outT = (jnp.dot(l1aT_ref[...], x1s[g],
                        preferred_element_type=jnp.float32)
                + jnp.dot(l1bT_ref[...], x2T,
                          preferred_element_type=jnp.float32)
                + bl1c_ref[...])
        pool_ref[g] = jnp.max(outT, axis=1, keepdims=True).T


def _head_body(x_ref, w1_ref, b1_ref, w2_ref, b2_ref, w3_ref, b3_ref, o_ref):
    x = x_ref[...]
    h = jnp.maximum(jnp.dot(x, w1_ref[...], preferred_element_type=jnp.float32)
                    + b1_ref[...], 0.0)
    h = jnp.maximum(jnp.dot(h, w2_ref[...], preferred_element_type=jnp.float32)
                    + b2_ref[...], 0.0)
    z = (jnp.dot(h, w3_ref[...], preferred_element_type=jnp.float32)
         + b3_ref[...])
    zm = z - jnp.max(z, axis=1, keepdims=True)
    o_ref[...] = zm - jnp.log(jnp.sum(jnp.exp(zm), axis=1, keepdims=True))


def _full(shape):
    return pl.BlockSpec(shape, lambda b: (0,) * len(shape))


def _perb(shape):
    return pl.BlockSpec((1,) + shape, lambda b: (b,) + (0,) * len(shape))


def _perg1(shape):
    return pl.BlockSpec((_G1,) + shape, lambda b: (b,) + (0,) * len(shape))


def _perg3(shape):
    return pl.BlockSpec((_G3,) + shape, lambda b: (b,) + (0,) * len(shape))


def kernel(pos, batch, params):
    p = params
    posT = jnp.transpose(pos.reshape(_B, _P, 3), (0, 2, 1))
    w1aT = p['c1_w1'][:3].T
    w1bT = p['c1_w1'][3:].T
    b1c = p['c1_b1'].reshape(64, 1)
    g1 = p['c1_g1'].reshape(1, 64)
    be1 = p['c1_be1'].reshape(1, 64)
    w2T = p['c1_w2'].T
    b2c = p['c1_b2'].reshape(64, 1)
    g2 = p['c1_g2'].reshape(1, 64)
    be2 = p['c1_be2'].reshape(1, 64)
    w3T = p['c1_w3'].T
    b3c = p['c1_b3'].reshape(64, 1)
    w2aT = p['c2_w1'][:64].T
    w2bT = p['c2_w1'][64:].T
    c2bc = p['c2_b1'].reshape(128, 1)
    l1aT = p['lin1_w'][:64].T
    l1bT = p['lin1_w'][64:].T
    bl1c = p['lin1_b'].reshape(1024, 1)

    f32 = jnp.float32
    h1, s1, q1 = pl.pallas_call(
        _c1_body,
        grid=(_B // _G1,),
        in_specs=[_perg1((3, _P)), _full((64, 3)), _full((64, 3)),
                  _full((64, 1))],
        out_specs=(_perg1((64, _K * _P)), _perg1((1, 64)), _perg1((1, 64))),
        out_shape=(jax.ShapeDtypeStruct((_B, 64, _K * _P), f32),
                   jax.ShapeDtypeStruct((_B, 1, 64), f32),
                   jax.ShapeDtypeStruct((_B, 1, 64), f32)),
    )(posT, w1aT, w1bT, b1c)

    h2, s2, q2 = pl.pallas_call(
        _c2_body,
        grid=(_B,),
        in_specs=[_perb((64, _K * _P)), _full((_B, 1, 64)),
                  _full((_B, 1, 64)), _full((1, 64)), _full((1, 64)),
                  _full((64, 64)), _full((64, 1))],
        out_specs=(_perb((64, _K * _P)), _perb((1, 64)), _perb((1, 64))),
        out_shape=(jax.ShapeDtypeStruct((_B, 64, _K * _P), f32),
                   jax.ShapeDtypeStruct((_B, 1, 64), f32),
                   jax.ShapeDtypeStruct((_B, 1, 64), f32)),
    )(h1, s1, q1, g1, be1, w2T, b2c)

    pooled = pl.pallas_call(
        _c3_body,
        grid=(_B // _G3,),
        in_specs=[_perg3((64, _K * _P)), _full((_B, 1, 64)),
                  _full((_B, 1, 64)), _full((1, 64)), _full((1, 64)),
                  _full((64, 64)), _full((64, 1)), _full((128, 64)),
                  _full((128, 64)), _full((128, 1)), _full((1024, 64)),
                  _full((1024, 128)), _full((1024, 1))],
        out_specs=_perg3((1, 1024)),
        out_shape=jax.ShapeDtypeStruct((_B, 1, 1024), f32),
    )(h2, s2, q2, g2, be2, w3T, b3c, w2aT, w2bT, c2bc, l1aT, l1bT, bl1c)

    out = pl.pallas_call(
        _head_body,
        in_specs=[pl.BlockSpec((_B, 1024), lambda: (0, 0)),
                  pl.BlockSpec((1024, 512), lambda: (0, 0)),
                  pl.BlockSpec((1, 512), lambda: (0, 0)),
                  pl.BlockSpec((512, 256), lambda: (0, 0)),
                  pl.BlockSpec((1, 256), lambda: (0, 0)),
                  pl.BlockSpec((256, 40), lambda: (0, 0)),
                  pl.BlockSpec((1, 40), lambda: (0, 0))],
        out_specs=pl.BlockSpec((_B, 40), lambda: (0, 0)),
        out_shape=jax.ShapeDtypeStruct((_B, 40), f32),
    )(pooled.reshape(_B, 1024), p['m_w1'], p['m_b1'].reshape(1, 512),
      p['m_w2'], p['m_b2'].reshape(1, 256), p['m_w3'],
      p['m_b3'].reshape(1, 40))
    return out
